# baseline (speedup 1.0000x reference)
import functools

import jax
import jax.numpy as jnp
from jax import lax
from jax.experimental import pallas as pl
from jax.experimental.pallas import tpu as pltpu
from jax.experimental.pallas import tpu_sc as plsc

B, N, K = 4, 1024, 16
RADII = (0.1, 0.3, 0.6)
EPS = 1e-3
L = 16
NW = 32
ROWS_PER_W = (B * N) // NW
CHUNK = 16
QB = 256
NBLK = 64


def _swish(x):
    return x * jax.nn.sigmoid(x)


def _bn(x, g, b, m, v):
    scale = g * lax.rsqrt(v + EPS)
    return x * scale + (b - m * scale)


def _tnet_body(inp_ref, tcw, tcb, g1, b1, m1, v1, d1w, d1b, g2, b2, m2, v2,
               d2w, d2b, pct_ref):
    x = inp_ref[0]
    h = jnp.dot(x, tcw[...]) + tcb[...]
    h = _swish(_bn(h, g1[...], b1[...], m1[...], v1[...]))
    hm = jnp.max(h, axis=0, keepdims=True)
    h2 = jnp.dot(hm, d1w[...]) + d1b[...]
    h2 = _swish(_bn(h2, g2[...], b2[...], m2[...], v2[...]))
    x9 = jnp.dot(h2, d2w[...]) + d2b[...]
    t = jnp.concatenate([x9[:, 0:3], x9[:, 3:6], x9[:, 6:9]], axis=0)
    pct_ref[0] = jnp.dot(x, t)


def _tnet(inputs, tcw, tcb, g1, b1, m1, v1, d1w, d1b, g2, b2, m2, v2, d2w, d2b):
    full = lambda *s: pl.BlockSpec(s, lambda i: (0,) * len(s))
    return pl.pallas_call(
        _tnet_body,
        grid=(B,),
        in_specs=[
            pl.BlockSpec((1, N, 3), lambda i: (i, 0, 0)),
            full(3, 128), full(128,), full(128,), full(128,), full(128,),
            full(128,), full(128, 128), full(128,), full(128,), full(128,),
            full(128,), full(128,), full(128, 9), full(9,),
        ],
        out_specs=pl.BlockSpec((1, N, 3), lambda i: (i, 0, 0)),
        out_shape=jax.ShapeDtypeStruct((B, N, 3), jnp.float32),
    )(inputs, tcw, tcb, g1, b1, m1, v1, d1w, d1b, g2, b2, m2, v2, d2w, d2b)


def _dist_body(pct_ref, pctt_ref, out_ref):
    q = pct_ref[0]
    p = pctt_ref[0]
    d0 = q[:, 0:1] - p[0:1, :]
    s = d0 * d0
    d1 = q[:, 1:2] - p[1:2, :]
    s = s + d1 * d1
    d2 = q[:, 2:3] - p[2:3, :]
    s = s + d2 * d2
    out_ref[0] = jnp.sqrt(jnp.maximum(s, 1e-12))


def _dists(pct, pct_t):
    return pl.pallas_call(
        _dist_body,
        grid=(B, N // QB),
        in_specs=[
            pl.BlockSpec((1, QB, 3), lambda b, q: (b, q, 0)),
            pl.BlockSpec((1, 3, N), lambda b, q: (b, 0, 0)),
        ],
        out_specs=pl.BlockSpec((1, QB, N), lambda b, q: (b, q, 0)),
        out_shape=jax.ShapeDtypeStruct((B, N, N), jnp.float32),
    )(pct, pct_t)


def _sc_select_body(dist_hbm, pct_hbm, feats_hbm, drows, pctv, fout,
                    ib0, ib1, ib2, ob0, ob1, ob2,
                    hitbuf_a, hitbuf_b, hitbuf_c, hitbuf_d):
    wid = lax.axis_index("s") * 2 + lax.axis_index("c")
    tiles_per_b = N // ROWS_PER_W
    b = wid // tiles_per_b
    n0 = (wid % tiles_per_b) * ROWS_PER_W
    pltpu.sync_copy(pct_hbm.at[b], pctv)

    lanes = lax.broadcasted_iota(jnp.int32, (L,), 0)
    outbufs = (ob0, ob1, ob2)

    def zrow(i, _):
        fout[i] = jnp.zeros((L,), jnp.float32)
        return 0
    lax.fori_loop(0, CHUNK * K, zrow, 0)

    def row_tail(rj, h, hitbuf):
        nh = (h + L - 1) // L

        def cond2(st):
            return (st[0] < nh) & ((st[1] < K) | (st[2] < K))

        def body2(st):
            w, i0, i1 = st
            hidx = hitbuf[pl.ds(w * L, L)]
            valid = (w * L + lanes) < h
            hidx = jnp.where(valid, hidx, lanes * 0)
            rowsp = lanes * 0 + rj
            dv = plsc.load_gather(drows, [rowsp, hidx])
            m0 = (dv <= RADII[0]) & valid
            m1 = (dv <= RADII[1]) & valid
            pc0 = plsc.all_reduce_population_count(m0)[0]
            pc1 = plsc.all_reduce_population_count(m1)[0]

            @pl.when(i0 < K)
            def _():
                plsc.store_compressed(ib0.at[pl.ds(i0, L)], hidx, mask=m0)

            @pl.when(i1 < K)
            def _():
                plsc.store_compressed(ib1.at[pl.ds(i1, L)], hidx, mask=m1)
            return (w + 1, i0 + pc0, i1 + pc1)

        z = jnp.int32(0)
        _, i0, i1 = lax.while_loop(cond2, body2, (z, z, z))

        def condo(st):
            return ((st[0] < N // L) &
                    ((st[1] < K) | (st[2] < K) | (st[3] < K)))

        def bodyo(st):
            w, o0, o1, o2 = st
            d = drows[rj, pl.ds(w * L, L)]
            news = []
            ocurs = (o0, o1, o2)
            for ri in range(3):
                m = jnp.logical_not(d <= RADII[ri])
                pc = plsc.all_reduce_population_count(m)[0]
                ocur = ocurs[ri]

                @pl.when(ocur < K)
                def _():
                    plsc.store_compressed(
                        outbufs[ri].at[pl.ds(ocur, L)],
                        lanes + w * L, mask=m)
                news.append(ocur + pc)
            return (w + 1, news[0], news[1], news[2])

        _, o0, o1, o2 = lax.while_loop(condo, bodyo, (z, z, z, z))

        ins = (i0, i1, h)
        srcbufs = (ib0, ib1, hitbuf)
        for ri in range(3):
            c = jnp.minimum(ins[ri], K)
            use_in = lanes < c
            jin = plsc.load_gather(srcbufs[ri], [lanes])
            jout = plsc.load_gather(outbufs[ri],
                                    [jnp.maximum(lanes - c, 0)])
            j = jnp.where(use_in, jin, jout)
            for c3 in range(3):
                vals = plsc.load_gather(pctv, [j * 3 + c3])
                col = jnp.full((L,), ri * 3 + c3, jnp.int32)
                plsc.store_scatter(fout, [rj * K + lanes, col], vals)

    def chunk_step(ci, _):
        row_base = n0 + ci * CHUNK
        pltpu.sync_copy(dist_hbm.at[b, pl.ds(row_base, CHUNK)], drows)

        def quad_step(rp, _2):
            rows = [rp * 4 + q for q in range(4)]
            hbufs = (hitbuf_a, hitbuf_b, hitbuf_c, hitbuf_d)

            def p1(v, carry):
                hs = list(carry)
                for q in range(4):
                    d = drows[rows[q], pl.ds(v * L, L)]
                    m = d <= RADII[2]
                    plsc.store_compressed(hbufs[q].at[pl.ds(hs[q], L)],
                                          lanes + v * L, mask=m)
                    hs[q] = hs[q] + plsc.all_reduce_population_count(m)[0]
                return tuple(hs)

            z = jnp.int32(0)
            hs = lax.fori_loop(0, N // L, p1, (z, z, z, z))
            for q in range(4):
                row_tail(rows[q], hs[q], hbufs[q])
            return 0

        lax.fori_loop(0, CHUNK // 4, quad_step, 0)
        pltpu.sync_copy(fout,
                        feats_hbm.at[b, pl.ds(row_base * K, CHUNK * K)])
        return 0

    lax.fori_loop(0, ROWS_PER_W // CHUNK, chunk_step, 0)


def _sc_select(dist, pct_flat):
    mesh = plsc.VectorSubcoreMesh(core_axis_name="c", subcore_axis_name="s",
                                  num_cores=2, num_subcores=16)
    f = functools.partial(
        pl.kernel,
        out_type=jax.ShapeDtypeStruct((B, N * K, L), jnp.float32),
        mesh=mesh,
        compiler_params=pltpu.CompilerParams(needs_layout_passes=False),
        scratch_types=[
            pltpu.VMEM((CHUNK, N), jnp.float32),
            pltpu.VMEM((N * 3,), jnp.float32),
            pltpu.VMEM((CHUNK * K, L), jnp.float32),
            pltpu.VMEM((2 * L,), jnp.int32),
            pltpu.VMEM((2 * L,), jnp.int32),
            pltpu.VMEM((2 * L,), jnp.int32),
            pltpu.VMEM((2 * L,), jnp.int32),
            pltpu.VMEM((2 * L,), jnp.int32),
            pltpu.VMEM((2 * L,), jnp.int32),
            pltpu.VMEM((N + L,), jnp.int32),
            pltpu.VMEM((N + L,), jnp.int32),
            pltpu.VMEM((N + L,), jnp.int32),
            pltpu.VMEM((N + L,), jnp.int32),
        ],
    )(_sc_select_body)
    return f(dist, pct_flat)


def _conv_body(feats_ref, c1w, c1b, g1, b1, m1, v1, rw, rb,
               w0, b0, g0, bb0, m0, v0, w1, bb1, g11, b11, m11, v11, out_ref):
    x = feats_ref[0]
    f = jnp.dot(x, c1w[...]) + c1b[...]
    f = _swish(_bn(f, g1[...], b1[...], m1[...], v1[...]))
    res = jnp.dot(f, rw[...]) + rb[...]
    f = f + res
    f = jnp.dot(f, w0[...]) + b0[...]
    f = _swish(_bn(f, g0[...], bb0[...], m0[...], v0[...]))
    f = jnp.dot(f, w1[...]) + bb1[...]
    f = _swish(_bn(f, g11[...], b11[...], m11[...], v11[...]))
    m = jnp.max(f.reshape(NBLK, K, 128), axis=0)
    nb = pl.program_id(1)

    @pl.when(nb == 0)
    def _():
        out_ref[0] = m

    @pl.when(nb != 0)
    def _():
        out_ref[0] = jnp.maximum(out_ref[0], m)


def _conv(feats, c1wp, c1b, g1, b1, m1, v1, rw, rb,
          w0, b0, g0, bb0, m0, v0, w1, bb1, g11, b11, m11, v11):
    full = lambda *s: pl.BlockSpec(s, lambda i, j: (0,) * len(s))
    return pl.pallas_call(
        _conv_body,
        grid=(B, N // NBLK),
        in_specs=[
            pl.BlockSpec((1, NBLK * K, L), lambda bb, nb: (bb, nb, 0)),
            full(L, 256), full(256,), full(256,), full(256,), full(256,),
            full(256,), full(256, 256), full(256,),
            full(256, 256), full(256,), full(256,), full(256,), full(256,),
            full(256,), full(256, 128), full(128,), full(128,), full(128,),
            full(128,), full(128,),
        ],
        out_specs=pl.BlockSpec((1, K, 128), lambda bb, nb: (bb, 0, 0)),
        out_shape=jax.ShapeDtypeStruct((B, K, 128), jnp.float32),
    )(feats, c1wp, c1b, g1, b1, m1, v1, rw, rb,
      w0, b0, g0, bb0, m0, v0, w1, bb1, g11, b11, m11, v11)


def kernel(inputs, t_conv_w, t_conv_b, t_bn1_g, t_bn1_b, t_bn1_m, t_bn1_v,
           t_d1_w, t_d1_b, t_bn2_g, t_bn2_b, t_bn2_m, t_bn2_v, t_d2_w, t_d2_b,
           c1_w, c1_b, bn1_g, bn1_b, bn1_m, bn1_v, res_w, res_b,
           blk0_w, blk0_b, blk0_bn_g, blk0_bn_b, blk0_bn_m, blk0_bn_v,
           blk1_w, blk1_b, blk1_bn_g, blk1_bn_b, blk1_bn_m, blk1_bn_v):
    pct = _tnet(inputs, t_conv_w, t_conv_b, t_bn1_g, t_bn1_b, t_bn1_m,
                t_bn1_v, t_d1_w, t_d1_b, t_bn2_g, t_bn2_b, t_bn2_m, t_bn2_v,
                t_d2_w, t_d2_b)
    pct_t = jnp.swapaxes(pct, 1, 2)
    dist = _dists(pct, pct_t)
    feats = _sc_select(dist, pct.reshape(B, N * 3))
    c1wp = jnp.concatenate(
        [c1_w, jnp.zeros((L - 9, c1_w.shape[1]), jnp.float32)], axis=0)
    out = _conv(feats, c1wp, c1_b, bn1_g, bn1_b, bn1_m, bn1_v, res_w, res_b,
                blk0_w, blk0_b, blk0_bn_g, blk0_bn_b, blk0_bn_m, blk0_bn_v,
                blk1_w, blk1_b, blk1_bn_g, blk1_bn_b, blk1_bn_m, blk1_bn_v)
    return out.reshape(B, K * 128)


# speedup vs baseline: 1.0670x; 1.0670x over previous
import functools

import jax
import jax.numpy as jnp
from jax import lax
from jax.experimental import pallas as pl
from jax.experimental.pallas import tpu as pltpu
from jax.experimental.pallas import tpu_sc as plsc

B, N, K = 4, 1024, 16
RADII = (0.1, 0.3, 0.6)
EPS = 1e-3
L = 16
NW = 32
ROWS_PER_W = (B * N) // NW
CHUNK = 16
QB = 256
NBLK = 64


def _swish(x):
    return x * jax.nn.sigmoid(x)


def _bn(x, g, b, m, v):
    scale = g * lax.rsqrt(v + EPS)
    return x * scale + (b - m * scale)


def _tnet_body(inp_ref, tcw, tcb, g1, b1, m1, v1, d1w, d1b, g2, b2, m2, v2,
               d2w, d2b, pct_ref):
    x = inp_ref[0]
    h = jnp.dot(x, tcw[...]) + tcb[...]
    h = _swish(_bn(h, g1[...], b1[...], m1[...], v1[...]))
    hm = jnp.max(h, axis=0, keepdims=True)
    h2 = jnp.dot(hm, d1w[...]) + d1b[...]
    h2 = _swish(_bn(h2, g2[...], b2[...], m2[...], v2[...]))
    x9 = jnp.dot(h2, d2w[...]) + d2b[...]
    t = jnp.concatenate([x9[:, 0:3], x9[:, 3:6], x9[:, 6:9]], axis=0)
    pct_ref[0] = jnp.dot(x, t)


def _tnet(inputs, tcw, tcb, g1, b1, m1, v1, d1w, d1b, g2, b2, m2, v2, d2w, d2b):
    full = lambda *s: pl.BlockSpec(s, lambda i: (0,) * len(s))
    return pl.pallas_call(
        _tnet_body,
        grid=(B,),
        in_specs=[
            pl.BlockSpec((1, N, 3), lambda i: (i, 0, 0)),
            full(3, 128), full(128,), full(128,), full(128,), full(128,),
            full(128,), full(128, 128), full(128,), full(128,), full(128,),
            full(128,), full(128,), full(128, 9), full(9,),
        ],
        out_specs=pl.BlockSpec((1, N, 3), lambda i: (i, 0, 0)),
        out_shape=jax.ShapeDtypeStruct((B, N, 3), jnp.float32),
    )(inputs, tcw, tcb, g1, b1, m1, v1, d1w, d1b, g2, b2, m2, v2, d2w, d2b)


def _dist_body(pct_ref, pctt_ref, out_ref):
    q = pct_ref[0]
    p = pctt_ref[0]
    d0 = q[:, 0:1] - p[0:1, :]
    s = d0 * d0
    d1 = q[:, 1:2] - p[1:2, :]
    s = s + d1 * d1
    d2 = q[:, 2:3] - p[2:3, :]
    s = s + d2 * d2
    out_ref[0] = jnp.sqrt(jnp.maximum(s, 1e-12))


def _dists(pct, pct_t):
    return pl.pallas_call(
        _dist_body,
        grid=(B, N // QB),
        in_specs=[
            pl.BlockSpec((1, QB, 3), lambda b, q: (b, q, 0)),
            pl.BlockSpec((1, 3, N), lambda b, q: (b, 0, 0)),
        ],
        out_specs=pl.BlockSpec((1, QB, N), lambda b, q: (b, q, 0)),
        out_shape=jax.ShapeDtypeStruct((B, N, N), jnp.float32),
    )(pct, pct_t)


def _sc_select_body(dist_hbm, pct_hbm, feats_hbm, drows, pctv, fout,
                    ib0, ib1, ib2, ob0, ob1, ob2, hitbuf_a, hitbuf_b):
    wid = lax.axis_index("s") * 2 + lax.axis_index("c")
    tiles_per_b = N // ROWS_PER_W
    b = wid // tiles_per_b
    n0 = (wid % tiles_per_b) * ROWS_PER_W
    pltpu.sync_copy(pct_hbm.at[b], pctv)

    lanes = lax.broadcasted_iota(jnp.int32, (L,), 0)
    outbufs = (ob0, ob1, ob2)

    def zrow(i, _):
        fout[i] = jnp.zeros((L,), jnp.float32)
        return 0
    lax.fori_loop(0, CHUNK * K, zrow, 0)

    def row_tail(rj, h, hitbuf):
        nh = (h + L - 1) // L

        def cond2(st):
            return (st[0] < nh) & ((st[1] < K) | (st[2] < K))

        def body2(st):
            w, i0, i1 = st
            hidx = hitbuf[pl.ds(w * L, L)]
            valid = (w * L + lanes) < h
            hidx = jnp.where(valid, hidx, lanes * 0)
            rowsp = lanes * 0 + rj
            dv = plsc.load_gather(drows, [rowsp, hidx])
            m0 = (dv <= RADII[0]) & valid
            m1 = (dv <= RADII[1]) & valid
            pc0 = plsc.all_reduce_population_count(m0)[0]
            pc1 = plsc.all_reduce_population_count(m1)[0]

            @pl.when(i0 < K)
            def _():
                plsc.store_compressed(ib0.at[pl.ds(i0, L)], hidx, mask=m0)

            @pl.when(i1 < K)
            def _():
                plsc.store_compressed(ib1.at[pl.ds(i1, L)], hidx, mask=m1)
            return (w + 1, i0 + pc0, i1 + pc1)

        z = jnp.int32(0)
        _, i0, i1 = lax.while_loop(cond2, body2, (z, z, z))

        def condo(st):
            return ((st[0] < N // L) &
                    ((st[1] < K) | (st[2] < K) | (st[3] < K)))

        def bodyo(st):
            w, o0, o1, o2 = st
            d = drows[rj, pl.ds(w * L, L)]
            news = []
            ocurs = (o0, o1, o2)
            for ri in range(3):
                m = jnp.logical_not(d <= RADII[ri])
                pc = plsc.all_reduce_population_count(m)[0]
                ocur = ocurs[ri]

                @pl.when(ocur < K)
                def _():
                    plsc.store_compressed(
                        outbufs[ri].at[pl.ds(ocur, L)],
                        lanes + w * L, mask=m)
                news.append(ocur + pc)
            return (w + 1, news[0], news[1], news[2])

        _, o0, o1, o2 = lax.while_loop(condo, bodyo, (z, z, z, z))

        ins = (i0, i1, h)
        srcbufs = (ib0, ib1, hitbuf)
        for ri in range(3):
            c = jnp.minimum(ins[ri], K)
            use_in = lanes < c
            jin = plsc.load_gather(srcbufs[ri], [lanes])
            jout = plsc.load_gather(outbufs[ri],
                                    [jnp.maximum(lanes - c, 0)])
            j = jnp.where(use_in, jin, jout)
            for c3 in range(3):
                vals = plsc.load_gather(pctv, [j * 3 + c3])
                col = jnp.full((L,), ri * 3 + c3, jnp.int32)
                plsc.store_scatter(fout, [rj * K + lanes, col], vals)

    def chunk_step(ci, _):
        row_base = n0 + ci * CHUNK
        pltpu.sync_copy(dist_hbm.at[b, pl.ds(row_base, CHUNK)], drows)

        def pair_step(rp, _2):
            ra = rp * 2
            rb = ra + 1

            def p1(vb, carry):
                ha, hb = carry
                for u in range(2):
                    v = vb * 2 + u
                    da = drows[ra, pl.ds(v * L, L)]
                    db = drows[rb, pl.ds(v * L, L)]
                    ma = da <= RADII[2]
                    mb = db <= RADII[2]
                    plsc.store_compressed(hitbuf_a.at[pl.ds(ha, L)],
                                          lanes + v * L, mask=ma)
                    plsc.store_compressed(hitbuf_b.at[pl.ds(hb, L)],
                                          lanes + v * L, mask=mb)
                    ha = ha + plsc.all_reduce_population_count(ma)[0]
                    hb = hb + plsc.all_reduce_population_count(mb)[0]
                return (ha, hb)

            z = jnp.int32(0)
            ha, hb = lax.fori_loop(0, N // (L * 2), p1, (z, z))
            row_tail(ra, ha, hitbuf_a)
            row_tail(rb, hb, hitbuf_b)
            return 0

        lax.fori_loop(0, CHUNK // 2, pair_step, 0)
        pltpu.sync_copy(fout,
                        feats_hbm.at[b, pl.ds(row_base * K, CHUNK * K)])
        return 0

    lax.fori_loop(0, ROWS_PER_W // CHUNK, chunk_step, 0)


def _sc_select(dist, pct_flat):
    mesh = plsc.VectorSubcoreMesh(core_axis_name="c", subcore_axis_name="s",
                                  num_cores=2, num_subcores=16)
    f = functools.partial(
        pl.kernel,
        out_type=jax.ShapeDtypeStruct((B, N * K, L), jnp.float32),
        mesh=mesh,
        compiler_params=pltpu.CompilerParams(needs_layout_passes=False),
        scratch_types=[
            pltpu.VMEM((CHUNK, N), jnp.float32),
            pltpu.VMEM((N * 3,), jnp.float32),
            pltpu.VMEM((CHUNK * K, L), jnp.float32),
            pltpu.VMEM((2 * L,), jnp.int32),
            pltpu.VMEM((2 * L,), jnp.int32),
            pltpu.VMEM((2 * L,), jnp.int32),
            pltpu.VMEM((2 * L,), jnp.int32),
            pltpu.VMEM((2 * L,), jnp.int32),
            pltpu.VMEM((2 * L,), jnp.int32),
            pltpu.VMEM((N + L,), jnp.int32),
            pltpu.VMEM((N + L,), jnp.int32),
        ],
    )(_sc_select_body)
    return f(dist, pct_flat)


def _conv_body(feats_ref, c1w, c1b, rw, rb, w0, b0, w1, b1, out_ref):
    x = feats_ref[0]
    f = _swish(jnp.dot(x, c1w[...]) + c1b[...])
    res = jnp.dot(f, rw[...]) + rb[...]
    f = f + res
    f = _swish(jnp.dot(f, w0[...]) + b0[...])
    f = _swish(jnp.dot(f, w1[...]) + b1[...])
    m = jnp.max(f.reshape(NBLK, K, 128), axis=0)
    nb = pl.program_id(1)

    @pl.when(nb == 0)
    def _():
        out_ref[0] = m

    @pl.when(nb != 0)
    def _():
        out_ref[0] = jnp.maximum(out_ref[0], m)


def _conv(feats, c1wp, c1b, rw, rb, w0, b0, w1, b1):
    full = lambda *s: pl.BlockSpec(s, lambda i, j: (0,) * len(s))
    return pl.pallas_call(
        _conv_body,
        grid=(B, N // NBLK),
        in_specs=[
            pl.BlockSpec((1, NBLK * K, L), lambda bb, nb: (bb, nb, 0)),
            full(L, 256), full(256,), full(256, 256), full(256,),
            full(256, 256), full(256,), full(256, 128), full(128,),
        ],
        out_specs=pl.BlockSpec((1, K, 128), lambda bb, nb: (bb, 0, 0)),
        out_shape=jax.ShapeDtypeStruct((B, K, 128), jnp.float32),
    )(feats, c1wp, c1b, rw, rb, w0, b0, w1, b1)


def kernel(inputs, t_conv_w, t_conv_b, t_bn1_g, t_bn1_b, t_bn1_m, t_bn1_v,
           t_d1_w, t_d1_b, t_bn2_g, t_bn2_b, t_bn2_m, t_bn2_v, t_d2_w, t_d2_b,
           c1_w, c1_b, bn1_g, bn1_b, bn1_m, bn1_v, res_w, res_b,
           blk0_w, blk0_b, blk0_bn_g, blk0_bn_b, blk0_bn_m, blk0_bn_v,
           blk1_w, blk1_b, blk1_bn_g, blk1_bn_b, blk1_bn_m, blk1_bn_v):
    pct = _tnet(inputs, t_conv_w, t_conv_b, t_bn1_g, t_bn1_b, t_bn1_m,
                t_bn1_v, t_d1_w, t_d1_b, t_bn2_g, t_bn2_b, t_bn2_m, t_bn2_v,
                t_d2_w, t_d2_b)
    pct_t = jnp.swapaxes(pct, 1, 2)
    dist = _dists(pct, pct_t)
    feats = _sc_select(dist, pct.reshape(B, N * 3))
    c1wp = jnp.concatenate(
        [c1_w, jnp.zeros((L - 9, c1_w.shape[1]), jnp.float32)], axis=0)
    s1 = bn1_g * jax.lax.rsqrt(bn1_v + EPS)
    t1 = bn1_b - bn1_m * s1
    s0 = blk0_bn_g * jax.lax.rsqrt(blk0_bn_v + EPS)
    t0 = blk0_bn_b - blk0_bn_m * s0
    s2 = blk1_bn_g * jax.lax.rsqrt(blk1_bn_v + EPS)
    t2 = blk1_bn_b - blk1_bn_m * s2
    out = _conv(feats, c1wp * s1[None, :], c1_b * s1 + t1, res_w, res_b,
                blk0_w * s0[None, :], blk0_b * s0 + t0,
                blk1_w * s2[None, :], blk1_b * s2 + t2)
    return out.reshape(B, K * 128)


# speedup vs baseline: 1.1262x; 1.0555x over previous
import functools

import jax
import jax.numpy as jnp
from jax import lax
from jax.experimental import pallas as pl
from jax.experimental.pallas import tpu as pltpu
from jax.experimental.pallas import tpu_sc as plsc

B, N, K = 4, 1024, 16
RADII = (0.1, 0.3, 0.6)
EPS = 1e-3
L = 16
NW = 32
ROWS_PER_W = (B * N) // NW
CHUNK = 32
QB = 256
NBLK = 128


def _swish(x):
    return x * jax.nn.sigmoid(x)


def _bn(x, g, b, m, v):
    scale = g * lax.rsqrt(v + EPS)
    return x * scale + (b - m * scale)


def _tnet_body(inp_ref, tcw, tcb, g1, b1, m1, v1, d1w, d1b, g2, b2, m2, v2,
               d2w, d2b, pct_ref):
    x = inp_ref[0]
    h = jnp.dot(x, tcw[...]) + tcb[...]
    h = _swish(_bn(h, g1[...], b1[...], m1[...], v1[...]))
    hm = jnp.max(h, axis=0, keepdims=True)
    h2 = jnp.dot(hm, d1w[...]) + d1b[...]
    h2 = _swish(_bn(h2, g2[...], b2[...], m2[...], v2[...]))
    x9 = jnp.dot(h2, d2w[...]) + d2b[...]
    t = jnp.concatenate([x9[:, 0:3], x9[:, 3:6], x9[:, 6:9]], axis=0)
    pct_ref[0] = jnp.dot(x, t)


def _tnet(inputs, tcw, tcb, g1, b1, m1, v1, d1w, d1b, g2, b2, m2, v2, d2w, d2b):
    full = lambda *s: pl.BlockSpec(s, lambda i: (0,) * len(s))
    return pl.pallas_call(
        _tnet_body,
        grid=(B,),
        in_specs=[
            pl.BlockSpec((1, N, 3), lambda i: (i, 0, 0)),
            full(3, 128), full(128,), full(128,), full(128,), full(128,),
            full(128,), full(128, 128), full(128,), full(128,), full(128,),
            full(128,), full(128,), full(128, 9), full(9,),
        ],
        out_specs=pl.BlockSpec((1, N, 3), lambda i: (i, 0, 0)),
        out_shape=jax.ShapeDtypeStruct((B, N, 3), jnp.float32),
    )(inputs, tcw, tcb, g1, b1, m1, v1, d1w, d1b, g2, b2, m2, v2, d2w, d2b)


def _dist_body(pct_ref, pctt_ref, out_ref):
    q = pct_ref[0]
    p = pctt_ref[0]
    d0 = q[:, 0:1] - p[0:1, :]
    s = d0 * d0
    d1 = q[:, 1:2] - p[1:2, :]
    s = s + d1 * d1
    d2 = q[:, 2:3] - p[2:3, :]
    s = s + d2 * d2
    out_ref[0] = jnp.sqrt(jnp.maximum(s, 1e-12))


def _dists(pct, pct_t):
    return pl.pallas_call(
        _dist_body,
        grid=(B, N // QB),
        in_specs=[
            pl.BlockSpec((1, QB, 3), lambda b, q: (b, q, 0)),
            pl.BlockSpec((1, 3, N), lambda b, q: (b, 0, 0)),
        ],
        out_specs=pl.BlockSpec((1, QB, N), lambda b, q: (b, q, 0)),
        out_shape=jax.ShapeDtypeStruct((B, N, N), jnp.float32),
    )(pct, pct_t)


def _sc_select_body(dist_hbm, pct_hbm, feats_hbm, drows, pctv, fout,
                    ib0, ib1, ib2, ob0, ob1, ob2, hitbuf_a, hitbuf_b):
    wid = lax.axis_index("s") * 2 + lax.axis_index("c")
    tiles_per_b = N // ROWS_PER_W
    b = wid // tiles_per_b
    n0 = (wid % tiles_per_b) * ROWS_PER_W
    pltpu.sync_copy(pct_hbm.at[b], pctv)

    lanes = lax.broadcasted_iota(jnp.int32, (L,), 0)
    outbufs = (ob0, ob1, ob2)

    def zrow(i, _):
        fout[i] = jnp.zeros((L,), jnp.float32)
        return 0
    lax.fori_loop(0, CHUNK * K, zrow, 0)

    def row_tail(rj, h, hitbuf):
        nh = (h + L - 1) // L

        def cond2(st):
            return (st[0] < nh) & ((st[1] < K) | (st[2] < K))

        def body2(st):
            w, i0, i1 = st
            hidx = hitbuf[pl.ds(w * L, L)]
            valid = (w * L + lanes) < h
            hidx = jnp.where(valid, hidx, lanes * 0)
            rowsp = lanes * 0 + rj
            dv = plsc.load_gather(drows, [rowsp, hidx])
            m0 = (dv <= RADII[0]) & valid
            m1 = (dv <= RADII[1]) & valid
            pc0 = plsc.all_reduce_population_count(m0)[0]
            pc1 = plsc.all_reduce_population_count(m1)[0]

            @pl.when(i0 < K)
            def _():
                plsc.store_compressed(ib0.at[pl.ds(i0, L)], hidx, mask=m0)

            @pl.when(i1 < K)
            def _():
                plsc.store_compressed(ib1.at[pl.ds(i1, L)], hidx, mask=m1)
            return (w + 1, i0 + pc0, i1 + pc1)

        z = jnp.int32(0)
        _, i0, i1 = lax.while_loop(cond2, body2, (z, z, z))

        def condo(st):
            return ((st[0] < N // L) &
                    ((st[1] < K) | (st[2] < K) | (st[3] < K)))

        def bodyo(st):
            w, o0, o1, o2 = st
            d = drows[rj, pl.ds(w * L, L)]
            news = []
            ocurs = (o0, o1, o2)
            for ri in range(3):
                m = jnp.logical_not(d <= RADII[ri])
                pc = plsc.all_reduce_population_count(m)[0]
                ocur = ocurs[ri]

                @pl.when(ocur < K)
                def _():
                    plsc.store_compressed(
                        outbufs[ri].at[pl.ds(ocur, L)],
                        lanes + w * L, mask=m)
                news.append(ocur + pc)
            return (w + 1, news[0], news[1], news[2])

        _, o0, o1, o2 = lax.while_loop(condo, bodyo, (z, z, z, z))

        ins = (i0, i1, h)
        srcbufs = (ib0, ib1, hitbuf)
        for ri in range(3):
            c = jnp.minimum(ins[ri], K)
            use_in = lanes < c
            jin = plsc.load_gather(srcbufs[ri], [lanes])
            jout = plsc.load_gather(outbufs[ri],
                                    [jnp.maximum(lanes - c, 0)])
            j = jnp.where(use_in, jin, jout)
            for c3 in range(3):
                vals = plsc.load_gather(pctv, [j * 3 + c3])
                col = jnp.full((L,), ri * 3 + c3, jnp.int32)
                plsc.store_scatter(fout, [rj * K + lanes, col], vals)

    def chunk_step(ci, _):
        row_base = n0 + ci * CHUNK
        pltpu.sync_copy(dist_hbm.at[b, pl.ds(row_base, CHUNK)], drows)

        def pair_step(rp, _2):
            ra = rp * 2
            rb = ra + 1

            def p1(vb, carry):
                ha, hb = carry
                for u in range(2):
                    v = vb * 2 + u
                    da = drows[ra, pl.ds(v * L, L)]
                    db = drows[rb, pl.ds(v * L, L)]
                    ma = da <= RADII[2]
                    mb = db <= RADII[2]
                    plsc.store_compressed(hitbuf_a.at[pl.ds(ha, L)],
                                          lanes + v * L, mask=ma)
                    plsc.store_compressed(hitbuf_b.at[pl.ds(hb, L)],
                                          lanes + v * L, mask=mb)
                    ha = ha + plsc.all_reduce_population_count(ma)[0]
                    hb = hb + plsc.all_reduce_population_count(mb)[0]
                return (ha, hb)

            z = jnp.int32(0)
            ha, hb = lax.fori_loop(0, N // (L * 2), p1, (z, z))
            row_tail(ra, ha, hitbuf_a)
            row_tail(rb, hb, hitbuf_b)
            return 0

        lax.fori_loop(0, CHUNK // 2, pair_step, 0)
        pltpu.sync_copy(fout,
                        feats_hbm.at[b, pl.ds(row_base * K, CHUNK * K)])
        return 0

    lax.fori_loop(0, ROWS_PER_W // CHUNK, chunk_step, 0)


def _sc_select(dist, pct_flat):
    mesh = plsc.VectorSubcoreMesh(core_axis_name="c", subcore_axis_name="s",
                                  num_cores=2, num_subcores=16)
    f = functools.partial(
        pl.kernel,
        out_type=jax.ShapeDtypeStruct((B, N * K, L), jnp.float32),
        mesh=mesh,
        compiler_params=pltpu.CompilerParams(needs_layout_passes=False),
        scratch_types=[
            pltpu.VMEM((CHUNK, N), jnp.float32),
            pltpu.VMEM((N * 3,), jnp.float32),
            pltpu.VMEM((CHUNK * K, L), jnp.float32),
            pltpu.VMEM((2 * L,), jnp.int32),
            pltpu.VMEM((2 * L,), jnp.int32),
            pltpu.VMEM((2 * L,), jnp.int32),
            pltpu.VMEM((2 * L,), jnp.int32),
            pltpu.VMEM((2 * L,), jnp.int32),
            pltpu.VMEM((2 * L,), jnp.int32),
            pltpu.VMEM((N + L,), jnp.int32),
            pltpu.VMEM((N + L,), jnp.int32),
        ],
    )(_sc_select_body)
    return f(dist, pct_flat)


def _conv_body(feats_ref, c1w, c1b, rw, rb, w0, b0, w1, b1, out_ref):
    x = feats_ref[0]
    f = _swish(jnp.dot(x, c1w[...]) + c1b[...])
    res = jnp.dot(f, rw[...]) + rb[...]
    f = f + res
    f = _swish(jnp.dot(f, w0[...]) + b0[...])
    f = _swish(jnp.dot(f, w1[...]) + b1[...])
    m = jnp.max(f.reshape(NBLK, K, 128), axis=0)
    nb = pl.program_id(1)

    @pl.when(nb == 0)
    def _():
        out_ref[0] = m

    @pl.when(nb != 0)
    def _():
        out_ref[0] = jnp.maximum(out_ref[0], m)


def _conv(feats, c1wp, c1b, rw, rb, w0, b0, w1, b1):
    full = lambda *s: pl.BlockSpec(s, lambda i, j: (0,) * len(s))
    return pl.pallas_call(
        _conv_body,
        grid=(B, N // NBLK),
        in_specs=[
            pl.BlockSpec((1, NBLK * K, L), lambda bb, nb: (bb, nb, 0)),
            full(L, 256), full(256,), full(256, 256), full(256,),
            full(256, 256), full(256,), full(256, 128), full(128,),
        ],
        out_specs=pl.BlockSpec((1, K, 128), lambda bb, nb: (bb, 0, 0)),
        out_shape=jax.ShapeDtypeStruct((B, K, 128), jnp.float32),
    )(feats, c1wp, c1b, rw, rb, w0, b0, w1, b1)


def kernel(inputs, t_conv_w, t_conv_b, t_bn1_g, t_bn1_b, t_bn1_m, t_bn1_v,
           t_d1_w, t_d1_b, t_bn2_g, t_bn2_b, t_bn2_m, t_bn2_v, t_d2_w, t_d2_b,
           c1_w, c1_b, bn1_g, bn1_b, bn1_m, bn1_v, res_w, res_b,
           blk0_w, blk0_b, blk0_bn_g, blk0_bn_b, blk0_bn_m, blk0_bn_v,
           blk1_w, blk1_b, blk1_bn_g, blk1_bn_b, blk1_bn_m, blk1_bn_v):
    pct = _tnet(inputs, t_conv_w, t_conv_b, t_bn1_g, t_bn1_b, t_bn1_m,
                t_bn1_v, t_d1_w, t_d1_b, t_bn2_g, t_bn2_b, t_bn2_m, t_bn2_v,
                t_d2_w, t_d2_b)
    pct_t = jnp.swapaxes(pct, 1, 2)
    dist = _dists(pct, pct_t)
    feats = _sc_select(dist, pct.reshape(B, N * 3))
    c1wp = jnp.concatenate(
        [c1_w, jnp.zeros((L - 9, c1_w.shape[1]), jnp.float32)], axis=0)
    s1 = bn1_g * jax.lax.rsqrt(bn1_v + EPS)
    t1 = bn1_b - bn1_m * s1
    s0 = blk0_bn_g * jax.lax.rsqrt(blk0_bn_v + EPS)
    t0 = blk0_bn_b - blk0_bn_m * s0
    s2 = blk1_bn_g * jax.lax.rsqrt(blk1_bn_v + EPS)
    t2 = blk1_bn_b - blk1_bn_m * s2
    out = _conv(feats, c1wp * s1[None, :], c1_b * s1 + t1, res_w, res_b,
                blk0_w * s0[None, :], blk0_b * s0 + t0,
                blk1_w * s2[None, :], blk1_b * s2 + t2)
    return out.reshape(B, K * 128)


# speedup vs baseline: 1.1551x; 1.0257x over previous
import functools

import jax
import jax.numpy as jnp
from jax import lax
from jax.experimental import pallas as pl
from jax.experimental.pallas import tpu as pltpu
from jax.experimental.pallas import tpu_sc as plsc

B, N, K = 4, 1024, 16
RADII = (0.1, 0.3, 0.6)
EPS = 1e-3
L = 16
NW = 32
ROWS_PER_W = (B * N) // NW
CHUNK = 32
QB = 256
NBLK = 256


def _swish(x):
    return x * jax.nn.sigmoid(x)


def _bn(x, g, b, m, v):
    scale = g * lax.rsqrt(v + EPS)
    return x * scale + (b - m * scale)


def _tnet_body(inp_ref, tcw, tcb, g1, b1, m1, v1, d1w, d1b, g2, b2, m2, v2,
               d2w, d2b, pct_ref):
    x = inp_ref[0]
    h = jnp.dot(x, tcw[...]) + tcb[...]
    h = _swish(_bn(h, g1[...], b1[...], m1[...], v1[...]))
    hm = jnp.max(h, axis=0, keepdims=True)
    h2 = jnp.dot(hm, d1w[...]) + d1b[...]
    h2 = _swish(_bn(h2, g2[...], b2[...], m2[...], v2[...]))
    x9 = jnp.dot(h2, d2w[...]) + d2b[...]
    t = jnp.concatenate([x9[:, 0:3], x9[:, 3:6], x9[:, 6:9]], axis=0)
    pct_ref[0] = jnp.dot(x, t)


def _tnet(inputs, tcw, tcb, g1, b1, m1, v1, d1w, d1b, g2, b2, m2, v2, d2w, d2b):
    full = lambda *s: pl.BlockSpec(s, lambda i: (0,) * len(s))
    return pl.pallas_call(
        _tnet_body,
        grid=(B,),
        in_specs=[
            pl.BlockSpec((1, N, 3), lambda i: (i, 0, 0)),
            full(3, 128), full(128,), full(128,), full(128,), full(128,),
            full(128,), full(128, 128), full(128,), full(128,), full(128,),
            full(128,), full(128,), full(128, 9), full(9,),
        ],
        out_specs=pl.BlockSpec((1, N, 3), lambda i: (i, 0, 0)),
        out_shape=jax.ShapeDtypeStruct((B, N, 3), jnp.float32),
    )(inputs, tcw, tcb, g1, b1, m1, v1, d1w, d1b, g2, b2, m2, v2, d2w, d2b)


def _dist_body(pct_ref, pctt_ref, out_ref):
    q = pct_ref[0]
    p = pctt_ref[0]
    d0 = q[:, 0:1] - p[0:1, :]
    s = d0 * d0
    d1 = q[:, 1:2] - p[1:2, :]
    s = s + d1 * d1
    d2 = q[:, 2:3] - p[2:3, :]
    s = s + d2 * d2
    out_ref[0] = jnp.sqrt(jnp.maximum(s, 1e-12))


def _dists(pct, pct_t):
    return pl.pallas_call(
        _dist_body,
        grid=(B, N // QB),
        in_specs=[
            pl.BlockSpec((1, QB, 3), lambda b, q: (b, q, 0)),
            pl.BlockSpec((1, 3, N), lambda b, q: (b, 0, 0)),
        ],
        out_specs=pl.BlockSpec((1, QB, N), lambda b, q: (b, q, 0)),
        out_shape=jax.ShapeDtypeStruct((B, N, N), jnp.float32),
    )(pct, pct_t)


def _sc_select_body(dist_hbm, pct_hbm, feats_hbm, drows, pctv, fout,
                    ib0, ib1, ib2, ob0, ob1, ob2, hitbuf_a, hitbuf_b):
    wid = lax.axis_index("s") * 2 + lax.axis_index("c")
    tiles_per_b = N // ROWS_PER_W
    b = wid // tiles_per_b
    n0 = (wid % tiles_per_b) * ROWS_PER_W
    pltpu.sync_copy(pct_hbm.at[b], pctv)

    lanes = lax.broadcasted_iota(jnp.int32, (L,), 0)
    outbufs = (ob0, ob1, ob2)

    def zrow(i, _):
        fout[i] = jnp.zeros((L,), jnp.float32)
        return 0
    lax.fori_loop(0, CHUNK * K, zrow, 0)

    def row_tail(rj, h, hitbuf):
        nh = (h + L - 1) // L

        def cond2(st):
            return (st[0] < nh) & ((st[1] < K) | (st[2] < K))

        def body2(st):
            w, i0, i1 = st
            hidx = hitbuf[pl.ds(w * L, L)]
            valid = (w * L + lanes) < h
            hidx = jnp.where(valid, hidx, lanes * 0)
            rowsp = lanes * 0 + rj
            dv = plsc.load_gather(drows, [rowsp, hidx])
            m0 = (dv <= RADII[0]) & valid
            m1 = (dv <= RADII[1]) & valid
            pc0 = plsc.all_reduce_population_count(m0)[0]
            pc1 = plsc.all_reduce_population_count(m1)[0]

            @pl.when(i0 < K)
            def _():
                plsc.store_compressed(ib0.at[pl.ds(i0, L)], hidx, mask=m0)

            @pl.when(i1 < K)
            def _():
                plsc.store_compressed(ib1.at[pl.ds(i1, L)], hidx, mask=m1)
            return (w + 1, i0 + pc0, i1 + pc1)

        z = jnp.int32(0)
        _, i0, i1 = lax.while_loop(cond2, body2, (z, z, z))

        def condo(st):
            return ((st[0] < N // L) &
                    ((st[1] < K) | (st[2] < K) | (st[3] < K)))

        def bodyo(st):
            w, o0, o1, o2 = st
            d = drows[rj, pl.ds(w * L, L)]
            news = []
            ocurs = (o0, o1, o2)
            for ri in range(3):
                m = jnp.logical_not(d <= RADII[ri])
                pc = plsc.all_reduce_population_count(m)[0]
                ocur = ocurs[ri]

                @pl.when(ocur < K)
                def _():
                    plsc.store_compressed(
                        outbufs[ri].at[pl.ds(ocur, L)],
                        lanes + w * L, mask=m)
                news.append(ocur + pc)
            return (w + 1, news[0], news[1], news[2])

        _, o0, o1, o2 = lax.while_loop(condo, bodyo, (z, z, z, z))

        ins = (i0, i1, h)
        srcbufs = (ib0, ib1, hitbuf)
        for ri in range(3):
            c = jnp.minimum(ins[ri], K)
            use_in = lanes < c
            jin = plsc.load_gather(srcbufs[ri], [lanes])
            jout = plsc.load_gather(outbufs[ri],
                                    [jnp.maximum(lanes - c, 0)])
            j = jnp.where(use_in, jin, jout)
            for c3 in range(3):
                vals = plsc.load_gather(pctv, [j * 3 + c3])
                col = jnp.full((L,), ri * 3 + c3, jnp.int32)
                plsc.store_scatter(fout, [rj * K + lanes, col], vals)

    def chunk_step(ci, _):
        row_base = n0 + ci * CHUNK
        pltpu.sync_copy(dist_hbm.at[b, pl.ds(row_base, CHUNK)], drows)

        def pair_step(rp, _2):
            ra = rp * 2
            rb = ra + 1

            def p1(vb, carry):
                ha, hb = carry
                for u in range(2):
                    v = vb * 2 + u
                    da = drows[ra, pl.ds(v * L, L)]
                    db = drows[rb, pl.ds(v * L, L)]
                    ma = da <= RADII[2]
                    mb = db <= RADII[2]
                    plsc.store_compressed(hitbuf_a.at[pl.ds(ha, L)],
                                          lanes + v * L, mask=ma)
                    plsc.store_compressed(hitbuf_b.at[pl.ds(hb, L)],
                                          lanes + v * L, mask=mb)
                    ha = ha + plsc.all_reduce_population_count(ma)[0]
                    hb = hb + plsc.all_reduce_population_count(mb)[0]
                return (ha, hb)

            z = jnp.int32(0)
            ha, hb = lax.fori_loop(0, N // (L * 2), p1, (z, z))
            row_tail(ra, ha, hitbuf_a)
            row_tail(rb, hb, hitbuf_b)
            return 0

        lax.fori_loop(0, CHUNK // 2, pair_step, 0)
        pltpu.sync_copy(fout,
                        feats_hbm.at[b, pl.ds(row_base * K, CHUNK * K)])
        return 0

    lax.fori_loop(0, ROWS_PER_W // CHUNK, chunk_step, 0)


def _sc_select(dist, pct_flat):
    mesh = plsc.VectorSubcoreMesh(core_axis_name="c", subcore_axis_name="s",
                                  num_cores=2, num_subcores=16)
    f = functools.partial(
        pl.kernel,
        out_type=jax.ShapeDtypeStruct((B, N * K, L), jnp.float32),
        mesh=mesh,
        compiler_params=pltpu.CompilerParams(needs_layout_passes=False),
        scratch_types=[
            pltpu.VMEM((CHUNK, N), jnp.float32),
            pltpu.VMEM((N * 3,), jnp.float32),
            pltpu.VMEM((CHUNK * K, L), jnp.float32),
            pltpu.VMEM((2 * L,), jnp.int32),
            pltpu.VMEM((2 * L,), jnp.int32),
            pltpu.VMEM((2 * L,), jnp.int32),
            pltpu.VMEM((2 * L,), jnp.int32),
            pltpu.VMEM((2 * L,), jnp.int32),
            pltpu.VMEM((2 * L,), jnp.int32),
            pltpu.VMEM((N + L,), jnp.int32),
            pltpu.VMEM((N + L,), jnp.int32),
        ],
    )(_sc_select_body)
    return f(dist, pct_flat)


def _conv_body(feats_ref, c1w, c1b, rw, rb, w0, b0, w1, b1, out_ref):
    x = feats_ref[0]
    f = _swish(jnp.dot(x, c1w[...]) + c1b[...])
    res = jnp.dot(f, rw[...]) + rb[...]
    f = f + res
    f = _swish(jnp.dot(f, w0[...]) + b0[...])
    f = _swish(jnp.dot(f, w1[...]) + b1[...])
    m = jnp.max(f.reshape(NBLK, K, 128), axis=0)
    nb = pl.program_id(1)

    @pl.when(nb == 0)
    def _():
        out_ref[0] = m

    @pl.when(nb != 0)
    def _():
        out_ref[0] = jnp.maximum(out_ref[0], m)


def _conv(feats, c1wp, c1b, rw, rb, w0, b0, w1, b1):
    full = lambda *s: pl.BlockSpec(s, lambda i, j: (0,) * len(s))
    return pl.pallas_call(
        _conv_body,
        grid=(B, N // NBLK),
        in_specs=[
            pl.BlockSpec((1, NBLK * K, L), lambda bb, nb: (bb, nb, 0)),
            full(L, 256), full(256,), full(256, 256), full(256,),
            full(256, 256), full(256,), full(256, 128), full(128,),
        ],
        out_specs=pl.BlockSpec((1, K, 128), lambda bb, nb: (bb, 0, 0)),
        out_shape=jax.ShapeDtypeStruct((B, K, 128), jnp.float32),
    )(feats, c1wp, c1b, rw, rb, w0, b0, w1, b1)


def kernel(inputs, t_conv_w, t_conv_b, t_bn1_g, t_bn1_b, t_bn1_m, t_bn1_v,
           t_d1_w, t_d1_b, t_bn2_g, t_bn2_b, t_bn2_m, t_bn2_v, t_d2_w, t_d2_b,
           c1_w, c1_b, bn1_g, bn1_b, bn1_m, bn1_v, res_w, res_b,
           blk0_w, blk0_b, blk0_bn_g, blk0_bn_b, blk0_bn_m, blk0_bn_v,
           blk1_w, blk1_b, blk1_bn_g, blk1_bn_b, blk1_bn_m, blk1_bn_v):
    pct = _tnet(inputs, t_conv_w, t_conv_b, t_bn1_g, t_bn1_b, t_bn1_m,
                t_bn1_v, t_d1_w, t_d1_b, t_bn2_g, t_bn2_b, t_bn2_m, t_bn2_v,
                t_d2_w, t_d2_b)
    pct_t = jnp.swapaxes(pct, 1, 2)
    dist = _dists(pct, pct_t)
    feats = _sc_select(dist, pct.reshape(B, N * 3))
    c1wp = jnp.concatenate(
        [c1_w, jnp.zeros((L - 9, c1_w.shape[1]), jnp.float32)], axis=0)
    s1 = bn1_g * jax.lax.rsqrt(bn1_v + EPS)
    t1 = bn1_b - bn1_m * s1
    s0 = blk0_bn_g * jax.lax.rsqrt(blk0_bn_v + EPS)
    t0 = blk0_bn_b - blk0_bn_m * s0
    s2 = blk1_bn_g * jax.lax.rsqrt(blk1_bn_v + EPS)
    t2 = blk1_bn_b - blk1_bn_m * s2
    out = _conv(feats, c1wp * s1[None, :], c1_b * s1 + t1, res_w, res_b,
                blk0_w * s0[None, :], blk0_b * s0 + t0,
                blk1_w * s2[None, :], blk1_b * s2 + t2)
    return out.reshape(B, K * 128)


# speedup vs baseline: 1.1657x; 1.0091x over previous
import functools

import jax
import jax.numpy as jnp
from jax import lax
from jax.experimental import pallas as pl
from jax.experimental.pallas import tpu as pltpu
from jax.experimental.pallas import tpu_sc as plsc

B, N, K = 4, 1024, 16
RADII = (0.1, 0.3, 0.6)
EPS = 1e-3
L = 16
NW = 32
ROWS_PER_W = (B * N) // NW
CHUNK = 32
QB = 256
NBLK = 512


def _swish(x):
    return x * jax.nn.sigmoid(x)


def _bn(x, g, b, m, v):
    scale = g * lax.rsqrt(v + EPS)
    return x * scale + (b - m * scale)


def _tnet_body(inp_ref, tcw, tcb, g1, b1, m1, v1, d1w, d1b, g2, b2, m2, v2,
               d2w, d2b, pct_ref):
    x = inp_ref[0]
    h = jnp.dot(x, tcw[...]) + tcb[...]
    h = _swish(_bn(h, g1[...], b1[...], m1[...], v1[...]))
    hm = jnp.max(h, axis=0, keepdims=True)
    h2 = jnp.dot(hm, d1w[...]) + d1b[...]
    h2 = _swish(_bn(h2, g2[...], b2[...], m2[...], v2[...]))
    x9 = jnp.dot(h2, d2w[...]) + d2b[...]
    t = jnp.concatenate([x9[:, 0:3], x9[:, 3:6], x9[:, 6:9]], axis=0)
    pct_ref[0] = jnp.dot(x, t)


def _tnet(inputs, tcw, tcb, g1, b1, m1, v1, d1w, d1b, g2, b2, m2, v2, d2w, d2b):
    full = lambda *s: pl.BlockSpec(s, lambda i: (0,) * len(s))
    return pl.pallas_call(
        _tnet_body,
        grid=(B,),
        in_specs=[
            pl.BlockSpec((1, N, 3), lambda i: (i, 0, 0)),
            full(3, 128), full(128,), full(128,), full(128,), full(128,),
            full(128,), full(128, 128), full(128,), full(128,), full(128,),
            full(128,), full(128,), full(128, 9), full(9,),
        ],
        out_specs=pl.BlockSpec((1, N, 3), lambda i: (i, 0, 0)),
        out_shape=jax.ShapeDtypeStruct((B, N, 3), jnp.float32),
    )(inputs, tcw, tcb, g1, b1, m1, v1, d1w, d1b, g2, b2, m2, v2, d2w, d2b)


def _dist_body(pct_ref, pctt_ref, out_ref):
    q = pct_ref[0]
    p = pctt_ref[0]
    d0 = q[:, 0:1] - p[0:1, :]
    s = d0 * d0
    d1 = q[:, 1:2] - p[1:2, :]
    s = s + d1 * d1
    d2 = q[:, 2:3] - p[2:3, :]
    s = s + d2 * d2
    out_ref[0] = jnp.sqrt(jnp.maximum(s, 1e-12))


def _dists(pct, pct_t):
    return pl.pallas_call(
        _dist_body,
        grid=(B, N // QB),
        in_specs=[
            pl.BlockSpec((1, QB, 3), lambda b, q: (b, q, 0)),
            pl.BlockSpec((1, 3, N), lambda b, q: (b, 0, 0)),
        ],
        out_specs=pl.BlockSpec((1, QB, N), lambda b, q: (b, q, 0)),
        out_shape=jax.ShapeDtypeStruct((B, N, N), jnp.float32),
    )(pct, pct_t)


def _sc_select_body(dist_hbm, pct_hbm, feats_hbm, drows, pctv, fout,
                    ib0, ib1, ib2, ob0, ob1, ob2, hitbuf_a, hitbuf_b):
    wid = lax.axis_index("s") * 2 + lax.axis_index("c")
    tiles_per_b = N // ROWS_PER_W
    b = wid // tiles_per_b
    n0 = (wid % tiles_per_b) * ROWS_PER_W
    pltpu.sync_copy(pct_hbm.at[b], pctv)

    lanes = lax.broadcasted_iota(jnp.int32, (L,), 0)
    outbufs = (ob0, ob1, ob2)

    def zrow(i, _):
        fout[i] = jnp.zeros((L,), jnp.float32)
        return 0
    lax.fori_loop(0, CHUNK * K, zrow, 0)

    def row_tail(rj, h, hitbuf):
        nh = (h + L - 1) // L

        def cond2(st):
            return (st[0] < nh) & ((st[1] < K) | (st[2] < K))

        def body2(st):
            w, i0, i1 = st
            hidx = hitbuf[pl.ds(w * L, L)]
            valid = (w * L + lanes) < h
            hidx = jnp.where(valid, hidx, lanes * 0)
            rowsp = lanes * 0 + rj
            dv = plsc.load_gather(drows, [rowsp, hidx])
            m0 = (dv <= RADII[0]) & valid
            m1 = (dv <= RADII[1]) & valid
            pc0 = plsc.all_reduce_population_count(m0)[0]
            pc1 = plsc.all_reduce_population_count(m1)[0]

            @pl.when(i0 < K)
            def _():
                plsc.store_compressed(ib0.at[pl.ds(i0, L)], hidx, mask=m0)

            @pl.when(i1 < K)
            def _():
                plsc.store_compressed(ib1.at[pl.ds(i1, L)], hidx, mask=m1)
            return (w + 1, i0 + pc0, i1 + pc1)

        z = jnp.int32(0)
        _, i0, i1 = lax.while_loop(cond2, body2, (z, z, z))

        def condo(st):
            return ((st[0] < N // L) &
                    ((st[1] < K) | (st[2] < K) | (st[3] < K)))

        def bodyo(st):
            w, o0, o1, o2 = st
            d = drows[rj, pl.ds(w * L, L)]
            news = []
            ocurs = (o0, o1, o2)
            for ri in range(3):
                m = jnp.logical_not(d <= RADII[ri])
                pc = plsc.all_reduce_population_count(m)[0]
                ocur = ocurs[ri]

                @pl.when(ocur < K)
                def _():
                    plsc.store_compressed(
                        outbufs[ri].at[pl.ds(ocur, L)],
                        lanes + w * L, mask=m)
                news.append(ocur + pc)
            return (w + 1, news[0], news[1], news[2])

        _, o0, o1, o2 = lax.while_loop(condo, bodyo, (z, z, z, z))

        ins = (i0, i1, h)
        srcbufs = (ib0, ib1, hitbuf)
        for ri in range(3):
            c = jnp.minimum(ins[ri], K)
            use_in = lanes < c
            jin = plsc.load_gather(srcbufs[ri], [lanes])
            jout = plsc.load_gather(outbufs[ri],
                                    [jnp.maximum(lanes - c, 0)])
            j = jnp.where(use_in, jin, jout)
            for c3 in range(3):
                vals = plsc.load_gather(pctv, [j * 3 + c3])
                col = jnp.full((L,), ri * 3 + c3, jnp.int32)
                plsc.store_scatter(fout, [rj * K + lanes, col], vals)

    def chunk_step(ci, _):
        row_base = n0 + ci * CHUNK
        pltpu.sync_copy(dist_hbm.at[b, pl.ds(row_base, CHUNK)], drows)

        def pair_step(rp, _2):
            ra = rp * 2
            rb = ra + 1

            def p1(vb, carry):
                ha, hb = carry
                for u in range(2):
                    v = vb * 2 + u
                    da = drows[ra, pl.ds(v * L, L)]
                    db = drows[rb, pl.ds(v * L, L)]
                    ma = da <= RADII[2]
                    mb = db <= RADII[2]
                    plsc.store_compressed(hitbuf_a.at[pl.ds(ha, L)],
                                          lanes + v * L, mask=ma)
                    plsc.store_compressed(hitbuf_b.at[pl.ds(hb, L)],
                                          lanes + v * L, mask=mb)
                    ha = ha + plsc.all_reduce_population_count(ma)[0]
                    hb = hb + plsc.all_reduce_population_count(mb)[0]
                return (ha, hb)

            z = jnp.int32(0)
            ha, hb = lax.fori_loop(0, N // (L * 2), p1, (z, z))
            row_tail(ra, ha, hitbuf_a)
            row_tail(rb, hb, hitbuf_b)
            return 0

        lax.fori_loop(0, CHUNK // 2, pair_step, 0)
        pltpu.sync_copy(fout,
                        feats_hbm.at[b, pl.ds(row_base * K, CHUNK * K)])
        return 0

    lax.fori_loop(0, ROWS_PER_W // CHUNK, chunk_step, 0)


def _sc_select(dist, pct_flat):
    mesh = plsc.VectorSubcoreMesh(core_axis_name="c", subcore_axis_name="s",
                                  num_cores=2, num_subcores=16)
    f = functools.partial(
        pl.kernel,
        out_type=jax.ShapeDtypeStruct((B, N * K, L), jnp.float32),
        mesh=mesh,
        compiler_params=pltpu.CompilerParams(needs_layout_passes=False),
        scratch_types=[
            pltpu.VMEM((CHUNK, N), jnp.float32),
            pltpu.VMEM((N * 3,), jnp.float32),
            pltpu.VMEM((CHUNK * K, L), jnp.float32),
            pltpu.VMEM((2 * L,), jnp.int32),
            pltpu.VMEM((2 * L,), jnp.int32),
            pltpu.VMEM((2 * L,), jnp.int32),
            pltpu.VMEM((2 * L,), jnp.int32),
            pltpu.VMEM((2 * L,), jnp.int32),
            pltpu.VMEM((2 * L,), jnp.int32),
            pltpu.VMEM((N + L,), jnp.int32),
            pltpu.VMEM((N + L,), jnp.int32),
        ],
    )(_sc_select_body)
    return f(dist, pct_flat)


def _conv_body(feats_ref, c1w, c1b, rw, rb, w0, b0, w1, b1, out_ref):
    x = feats_ref[0]
    f = _swish(jnp.dot(x, c1w[...]) + c1b[...])
    res = jnp.dot(f, rw[...]) + rb[...]
    f = f + res
    f = _swish(jnp.dot(f, w0[...]) + b0[...])
    f = _swish(jnp.dot(f, w1[...]) + b1[...])
    m = jnp.max(f.reshape(NBLK, K, 128), axis=0)
    nb = pl.program_id(1)

    @pl.when(nb == 0)
    def _():
        out_ref[0] = m

    @pl.when(nb != 0)
    def _():
        out_ref[0] = jnp.maximum(out_ref[0], m)


def _conv(feats, c1wp, c1b, rw, rb, w0, b0, w1, b1):
    full = lambda *s: pl.BlockSpec(s, lambda i, j: (0,) * len(s))
    return pl.pallas_call(
        _conv_body,
        grid=(B, N // NBLK),
        in_specs=[
            pl.BlockSpec((1, NBLK * K, L), lambda bb, nb: (bb, nb, 0)),
            full(L, 256), full(256,), full(256, 256), full(256,),
            full(256, 256), full(256,), full(256, 128), full(128,),
        ],
        out_specs=pl.BlockSpec((1, K, 128), lambda bb, nb: (bb, 0, 0)),
        out_shape=jax.ShapeDtypeStruct((B, K, 128), jnp.float32),
    )(feats, c1wp, c1b, rw, rb, w0, b0, w1, b1)


def kernel(inputs, t_conv_w, t_conv_b, t_bn1_g, t_bn1_b, t_bn1_m, t_bn1_v,
           t_d1_w, t_d1_b, t_bn2_g, t_bn2_b, t_bn2_m, t_bn2_v, t_d2_w, t_d2_b,
           c1_w, c1_b, bn1_g, bn1_b, bn1_m, bn1_v, res_w, res_b,
           blk0_w, blk0_b, blk0_bn_g, blk0_bn_b, blk0_bn_m, blk0_bn_v,
           blk1_w, blk1_b, blk1_bn_g, blk1_bn_b, blk1_bn_m, blk1_bn_v):
    pct = _tnet(inputs, t_conv_w, t_conv_b, t_bn1_g, t_bn1_b, t_bn1_m,
                t_bn1_v, t_d1_w, t_d1_b, t_bn2_g, t_bn2_b, t_bn2_m, t_bn2_v,
                t_d2_w, t_d2_b)
    pct_t = jnp.swapaxes(pct, 1, 2)
    dist = _dists(pct, pct_t)
    feats = _sc_select(dist, pct.reshape(B, N * 3))
    c1wp = jnp.concatenate(
        [c1_w, jnp.zeros((L - 9, c1_w.shape[1]), jnp.float32)], axis=0)
    s1 = bn1_g * jax.lax.rsqrt(bn1_v + EPS)
    t1 = bn1_b - bn1_m * s1
    s0 = blk0_bn_g * jax.lax.rsqrt(blk0_bn_v + EPS)
    t0 = blk0_bn_b - blk0_bn_m * s0
    s2 = blk1_bn_g * jax.lax.rsqrt(blk1_bn_v + EPS)
    t2 = blk1_bn_b - blk1_bn_m * s2
    out = _conv(feats, c1wp * s1[None, :], c1_b * s1 + t1, res_w, res_b,
                blk0_w * s0[None, :], blk0_b * s0 + t0,
                blk1_w * s2[None, :], blk1_b * s2 + t2)
    return out.reshape(B, K * 128)
